# baseline (speedup 1.0000x reference)
import functools

import jax
import jax.numpy as jnp
from jax import lax
from jax.experimental import pallas as pl
from jax.experimental.pallas import tpu as pltpu
from jax.experimental.pallas import tpu_sc as plsc

_B, _C_IN, _N_T = 4, 384, 196
_K, _D = 1024, 64
_P = _B * _N_T
_P_PAD = 1024


def _vq_idx_body(z_ref, w_ref, emb_ref, idx_ref):
    hi = jax.lax.Precision.HIGHEST
    wb = w_ref[...].astype(jnp.bfloat16)
    cols = []
    for b in range(_B):
        zb = z_ref[b].astype(jnp.bfloat16)
        cols.append(jnp.dot(wb, zb, preferred_element_type=jnp.float32))
    ze = jnp.concatenate(cols, axis=1)
    emb = emb_ref[...]
    g = jnp.dot(emb, ze, precision=hi, preferred_element_type=jnp.float32)
    x2 = jnp.sum(ze * ze, axis=0, keepdims=True)
    e2 = jnp.sum(emb * emb, axis=1, keepdims=True)
    d2 = jnp.maximum(x2 - 2.0 * g + e2, 0.0)
    snorm = jnp.sqrt(d2) / (jnp.sqrt(x2) + jnp.sqrt(e2))
    mval = jnp.min(snorm, axis=0, keepdims=True)
    row = jax.lax.broadcasted_iota(jnp.int32, (_K, _P), 0)
    midx = jnp.min(jnp.where(snorm == mval, row, _K), axis=0, keepdims=True)
    pad = jnp.zeros((1, _P_PAD - _P), jnp.int32)
    idx_ref[...] = jnp.concatenate([midx, pad], axis=1)


def _tr_body(rows_ref, out_ref):
    t = rows_ref[...].T
    for b in range(_B):
        out_ref[b] = t[:, b * _N_T:(b + 1) * _N_T]


@functools.cache
def _sc_gather():
    info = plsc.get_sparse_core_info()
    nc, ns = info.num_cores, info.num_subcores
    nw = nc * ns
    bpw = _P_PAD // nw
    mesh = plsc.VectorSubcoreMesh(core_axis_name="c", subcore_axis_name="s")

    @functools.partial(
        pl.kernel, mesh=mesh,
        compiler_params=pltpu.CompilerParams(use_tc_tiling_on_sc=False),
        out_type=jax.ShapeDtypeStruct((_P_PAD, _D), jnp.float32),
        scratch_types=[
            pltpu.VMEM((bpw,), jnp.int32),
            pltpu.VMEM((bpw, _D), jnp.float32),
            pltpu.SemaphoreType.DMA,
        ],
    )
    def gather(table_hbm, idx_hbm, out_hbm, idx_v, rows_v, sem):
        wid = lax.axis_index("s") * nc + lax.axis_index("c")
        base = wid * bpw
        pltpu.sync_copy(idx_hbm.at[pl.ds(base, bpw)], idx_v)
        pltpu.async_copy(table_hbm.at[idx_v], rows_v, sem).wait()
        pltpu.sync_copy(rows_v, out_hbm.at[pl.ds(base, bpw)])

    return gather


@functools.partial(jax.jit, static_argnames=())
def kernel(z, W, emb):
    idx2d = pl.pallas_call(
        _vq_idx_body,
        out_shape=jax.ShapeDtypeStruct((1, _P_PAD), jnp.int32),
    )(z, W, emb)
    rows = _sc_gather()(emb, idx2d.reshape(_P_PAD))
    return pl.pallas_call(
        _tr_body,
        out_shape=jax.ShapeDtypeStruct((_B, _D, _N_T), jnp.float32),
    )(rows)


# speedup vs baseline: 3.1348x; 3.1348x over previous
import functools

import jax
import jax.numpy as jnp
from jax.experimental import pallas as pl

_B, _C_IN, _N_T = 4, 384, 196
_K, _D = 1024, 64
_P = _B * _N_T


def _vq_body(z_ref, w_ref, emb_ref, out_ref):
    hi = jax.lax.Precision.HIGHEST
    wb = w_ref[...].astype(jnp.bfloat16)
    cols = []
    for b in range(_B):
        zb = z_ref[b].astype(jnp.bfloat16)
        cols.append(jnp.dot(wb, zb, preferred_element_type=jnp.float32))
    ze = jnp.concatenate(cols, axis=1)
    emb = emb_ref[...]
    g = jnp.dot(emb, ze, precision=hi, preferred_element_type=jnp.float32)
    x2 = jnp.sum(ze * ze, axis=0, keepdims=True)
    e2 = jnp.sum(emb * emb, axis=1, keepdims=True)
    d2 = jnp.maximum(x2 - 2.0 * g + e2, 0.0)
    snorm = jnp.sqrt(d2) / (jnp.sqrt(x2) + jnp.sqrt(e2))
    mval = jnp.min(snorm, axis=0, keepdims=True)
    row = jax.lax.broadcasted_iota(jnp.int32, (_K, _P), 0)
    midx = jnp.min(jnp.where(snorm == mval, row, _K), axis=0, keepdims=True)
    onehot = (row == midx).astype(jnp.bfloat16)
    embt = emb.T
    embt_hi = embt.astype(jnp.bfloat16)
    embt_lo = (embt - embt_hi.astype(jnp.float32)).astype(jnp.bfloat16)
    hilo = jnp.concatenate([embt_hi, embt_lo], axis=0)
    r = jnp.dot(hilo, onehot, preferred_element_type=jnp.float32)
    zq = r[:_D] + r[_D:]
    for b in range(_B):
        out_ref[b] = zq[:, b * _N_T:(b + 1) * _N_T]


@functools.partial(jax.jit, static_argnames=())
def kernel(z, W, emb):
    return pl.pallas_call(
        _vq_body,
        out_shape=jax.ShapeDtypeStruct((_B, _D, _N_T), jnp.float32),
    )(z, W, emb)
